# baseline (speedup 1.0000x reference)
import functools

import jax
import jax.numpy as jnp
from jax import lax
from jax.experimental import pallas as pl
from jax.experimental.pallas import tpu as pltpu
from jax.experimental.pallas import tpu_sc as plsc

H = 100
W = 200
N = H * W
NUM_CLASSES = 80
MAX_OUT = 300
NEG_INF = float("-inf")
BIG_F = 1.0e9

GR = 112
GP = 208
NPIX = GR * GP
ROW_VREGS = GP // 16
WRAD = 7
WROWS = 2 * WRAD + 1
OUT_ROWS = 304


def _cls_body(cls_ref, ids_ref, cmax_ref):
    x = cls_ref[...]
    m = jnp.max(x, axis=1, keepdims=True)
    lane = jax.lax.broadcasted_iota(jnp.int32, x.shape, 1)
    mi = jnp.min(jnp.where(x == m, lane, jnp.int32(2**30)), axis=1,
                 keepdims=True)
    ids_ref[...] = mi.astype(jnp.float32)
    cmax_ref[...] = m


def _prep_body(cm_ref, ctr_ref, reg_ref, cen_ref, thr_ref,
               s_ref, x1_ref, y1_ref, x2_ref, y2_ref,
               rmax_ref, ridx_ref):
    thr = thr_ref[0, 0]

    score = cm_ref[...] * ctr_ref[...]
    gi = jax.lax.broadcasted_iota(jnp.int32, (GR, GP), 0)
    ci = jax.lax.broadcasted_iota(jnp.int32, (GR, GP), 1)
    valid = (gi < H) & (ci < W)
    s = jnp.where((score > thr) & valid, score, NEG_INF)
    s_ref[...] = s

    rmax = jnp.max(s, axis=1, keepdims=True)
    minc = jnp.min(jnp.where(s == rmax, ci, jnp.int32(2**30)), axis=1,
                   keepdims=True)
    rowi = jax.lax.broadcasted_iota(jnp.int32, (GR, 1), 0)
    rmax_ref[...] = rmax
    ridx_ref[...] = (rowi * W + minc).astype(jnp.float32)

    x1_ref[...] = cen_ref[0] - reg_ref[0]
    y1_ref[...] = cen_ref[1] - reg_ref[1]
    x2_ref[...] = cen_ref[0] + reg_ref[2]
    y2_ref[...] = cen_ref[1] + reg_ref[3]


def _nms_sc_body(s_hbm, x1_hbm, y1_hbm, x2_hbm, y2_hbm, rm_hbm, ri_hbm,
                 par_hbm, out_hbm,
                 s_v, x1_v, y1_v, x2_v, y2_v,
                 par_v, out_v, rmax_v, ridx_v, sem):
    cid = lax.axis_index("c")
    sid = lax.axis_index("s")

    @pl.when(jnp.logical_and(cid == 0, sid == 0))
    def _body():
        cps = [pltpu.async_copy(src, dst, sem) for src, dst in
               [(s_hbm, s_v), (x1_hbm, x1_v), (y1_hbm, y1_v),
                (x2_hbm, x2_v), (y2_hbm, y2_v)]]
        pltpu.sync_copy(rm_hbm, rmax_v)
        pltpu.sync_copy(ri_hbm, ridx_v)
        pltpu.sync_copy(par_hbm, par_v)
        for cp in cps:
            cp.wait()

        li = lax.iota(jnp.int32, 16)
        lif = li.astype(jnp.float32)
        iou_thr = par_v[...]
        zero16 = jnp.zeros((16,), jnp.int32)

        def scan_row(r):
            g_orig = r * W
            rb = r * GP
            acc = s_v[pl.ds(rb, 16)]
            idxv = jnp.float32(1.0) * g_orig + lif
            for t in range(1, ROW_VREGS):
                sv = s_v[pl.ds(rb + t * 16, 16)]
                gt = sv > acc
                acc = jnp.where(gt, sv, acc)
                idxv = jnp.where(gt, (jnp.float32(1.0) * g_orig + t * 16) + lif,
                                 idxv)
            rm = jnp.max(acc)
            ri = jnp.min(jnp.where(acc == rm, idxv, BIG_F))
            return rm, ri

        def step(k, carry):
            rmxs = [rmax_v[pl.ds(t * 16, 16)] for t in range(7)]
            rixs = [ridx_v[pl.ds(t * 16, 16)] for t in range(7)]
            t01 = jnp.maximum(rmxs[0], rmxs[1])
            t23 = jnp.maximum(rmxs[2], rmxs[3])
            t45 = jnp.maximum(rmxs[4], rmxs[5])
            m = jnp.max(jnp.maximum(jnp.maximum(t01, t23),
                                    jnp.maximum(t45, rmxs[6])))
            cand = jnp.full((16,), BIG_F, jnp.float32)
            for t in range(7):
                cand = jnp.minimum(cand, jnp.where(rmxs[t] == m, rixs[t], BIG_F))
            idxf = jnp.min(cand)

            widx = idxf.astype(jnp.int32)
            wg = widx // W
            wc = widx % W
            wpos = wg * GP + wc
            wp16 = zero16 + wpos
            wx1 = plsc.load_gather(x1_v, [wp16])
            wy1 = plsc.load_gather(y1_v, [wp16])
            wx2 = plsc.load_gather(x2_v, [wp16])
            wy2 = plsc.load_gather(y2_v, [wp16])
            war = (wx2 - wx1) * (wy2 - wy1)

            valid = m > NEG_INF
            vf = jnp.where(valid, 1.0, 0.0)
            outrec = jnp.where(li == 0, m * vf,
                     jnp.where(li == 1, idxf, jnp.where(li == 2, vf, 0.0)))
            out_v[pl.ds(k * 16, 16)] = outrec

            rows = wg - WRAD + li
            inbv = (rows >= 0) & (rows < GR) & (li < WROWS)
            rowsc = jnp.clip(rows, 0, GR - 1)
            crm = plsc.load_gather(rmax_v, [rowsc])
            cri = plsc.load_gather(ridx_v, [rowsc])
            cpos = jnp.clip(rowsc * GP + cri.astype(jnp.int32) - rowsc * W,
                            0, NPIX - 1)
            cx1 = plsc.load_gather(x1_v, [cpos])
            cy1 = plsc.load_gather(y1_v, [cpos])
            cx2 = plsc.load_gather(x2_v, [cpos])
            cy2 = plsc.load_gather(y2_v, [cpos])
            cinter = (jnp.maximum(jnp.minimum(wx2, cx2) - jnp.maximum(wx1, cx1), 0.0)
                      * jnp.maximum(jnp.minimum(wy2, cy2) - jnp.maximum(wy1, cy1), 0.0))
            ciou = cinter / (war + (cx2 - cx1) * (cy2 - cy1) - cinter + 1e-8)
            dead0 = (((ciou > iou_thr) | (cpos == wpos)) & (crm > NEG_INF)
                     & inbv).astype(jnp.int32)

            cl = jnp.maximum(wc - WRAD, 0)
            t0 = cl // 16
            ta = t0 * 16
            tb = jnp.minimum(t0 + 1, ROW_VREGS - 1) * 16
            slots = []
            for j in range(WROWS):
                row = wg - WRAD + j
                inb = jnp.logical_and(row >= 0, row < GR)
                rb = jnp.clip(row, 0, GR - 1) * GP
                for toff in (ta, tb):
                    beg = rb + toff
                    sl = pl.ds(beg, 16)
                    slots.append((sl, s_v[sl], x1_v[sl], y1_v[sl],
                                  x2_v[sl], y2_v[sl], beg, inb))
            for sl, sv, bx1, by1, bx2, by2, beg, inb in slots:
                ix1 = jnp.maximum(wx1, bx1)
                iy1 = jnp.maximum(wy1, by1)
                ix2 = jnp.minimum(wx2, bx2)
                iy2 = jnp.minimum(wy2, by2)
                inter = (jnp.maximum(ix2 - ix1, 0.0)
                         * jnp.maximum(iy2 - iy1, 0.0))
                areab = (bx2 - bx1) * (by2 - by1)
                iou = inter / (war + areab - inter + 1e-8)
                supp = ((iou > iou_thr) & inb) | ((beg + li) == wpos)
                s_v[sl] = jnp.where(supp, NEG_INF, sv)

            def any_dead(dead):
                return jnp.max(dead) == 1

            def rescan_one(dead):
                lane = jnp.min(jnp.where(dead == 1, li, 16))
                r = jnp.clip(wg - WRAD + lane, 0, GR - 1)
                rm, ri = scan_row(r)
                plsc.store_scatter(rmax_v, [zero16 + r], zero16 * 0.0 + rm,
                                   mask=li == 0)
                plsc.store_scatter(ridx_v, [zero16 + r], zero16 * 0.0 + ri,
                                   mask=li == 0)
                return jnp.where(li == lane, 0, dead)

            lax.while_loop(any_dead, rescan_one, dead0)
            return carry

        lax.fori_loop(0, MAX_OUT, step, 0, unroll=False)

        pltpu.sync_copy(out_v, out_hbm)


@jax.jit
def _decode_nms(cls_t, ctr_t, reg_t, centers, score_threshold, iou_threshold):
    def pitch(a):
        padw = [(0, 0)] * (a.ndim - 2) + [(0, GR - H), (0, GP - W)]
        return jnp.pad(a, padw)

    idsf_c, cmax_c = pl.pallas_call(
        _cls_body,
        out_shape=[jax.ShapeDtypeStruct((N, 1), jnp.float32)] * 2,
        in_specs=[pl.BlockSpec(memory_space=pltpu.VMEM)],
        out_specs=[pl.BlockSpec(memory_space=pltpu.VMEM)] * 2,
    )(cls_t[0])

    cm_p = pitch(cmax_c.reshape(H, W))
    ctr_p = pitch(ctr_t[0].reshape(H, W))
    reg_p = pitch(reg_t[0].T.reshape(4, H, W))
    cen_p = pitch(centers.T.reshape(2, H, W))
    thr = jnp.asarray(score_threshold, jnp.float32).reshape(1, 1)

    grid2d = jax.ShapeDtypeStruct((GR, GP), jnp.float32)
    col1 = jax.ShapeDtypeStruct((GR, 1), jnp.float32)
    s0, x1, y1, x2, y2, rm0, ri0 = pl.pallas_call(
        _prep_body,
        out_shape=[grid2d] * 5 + [col1] * 2,
        in_specs=[
            pl.BlockSpec(memory_space=pltpu.VMEM),
            pl.BlockSpec(memory_space=pltpu.VMEM),
            pl.BlockSpec(memory_space=pltpu.VMEM),
            pl.BlockSpec(memory_space=pltpu.VMEM),
            pl.BlockSpec(memory_space=pltpu.SMEM),
        ],
        out_specs=[pl.BlockSpec(memory_space=pltpu.VMEM)] * 7,
    )(cm_p, ctr_p, reg_p, cen_p, thr)

    par = jnp.full((16,), jnp.asarray(iou_threshold, jnp.float32))

    nms = pl.kernel(
        _nms_sc_body,
        out_type=jax.ShapeDtypeStruct((OUT_ROWS * 16,), jnp.float32),
        mesh=plsc.VectorSubcoreMesh(core_axis_name="c", subcore_axis_name="s"),
        compiler_params=pltpu.CompilerParams(needs_layout_passes=False),
        scratch_types=[
            pltpu.VMEM((NPIX,), jnp.float32),
            pltpu.VMEM((NPIX,), jnp.float32),
            pltpu.VMEM((NPIX,), jnp.float32),
            pltpu.VMEM((NPIX,), jnp.float32),
            pltpu.VMEM((NPIX,), jnp.float32),
            pltpu.VMEM((16,), jnp.float32),
            pltpu.VMEM((OUT_ROWS * 16,), jnp.float32),
            pltpu.VMEM((GR,), jnp.float32),
            pltpu.VMEM((GR,), jnp.float32),
            pltpu.SemaphoreType.DMA,
        ],
    )

    out = nms(s0.reshape(NPIX), x1.reshape(NPIX), y1.reshape(NPIX),
              x2.reshape(NPIX), y2.reshape(NPIX),
              rm0.reshape(GR), ri0.reshape(GR), par)

    sel = out.reshape(OUT_ROWS, 16)[:MAX_OUT]
    out_scores = sel[:, 0][None]
    widx = sel[:, 1].astype(jnp.int32)
    vmask = sel[:, 2] > 0.0
    pidx = (widx // W) * GP + widx % W
    bx = jnp.stack([x1.reshape(NPIX)[pidx], y1.reshape(NPIX)[pidx],
                    x2.reshape(NPIX)[pidx], y2.reshape(NPIX)[pidx]], axis=-1)
    out_boxes = jnp.where(vmask[:, None], bx, 0.0)[None]
    out_ids = jnp.where(vmask, idsf_c.reshape(N)[jnp.clip(widx, 0, N - 1)]
                        .astype(jnp.int32), -1)[None]
    return out_boxes, out_scores, out_ids


def kernel(cls_target, ctr_target, reg_target, centers, score_threshold, iou_threshold):
    return _decode_nms(cls_target, ctr_target, reg_target, centers,
                       score_threshold, iou_threshold)


# speedup vs baseline: 1.1627x; 1.1627x over previous
import functools

import jax
import jax.numpy as jnp
from jax import lax
from jax.experimental import pallas as pl
from jax.experimental.pallas import tpu as pltpu
from jax.experimental.pallas import tpu_sc as plsc

H = 100
W = 200
N = H * W
NUM_CLASSES = 80
MAX_OUT = 300
NEG_INF = float("-inf")
BIG_F = 1.0e9

GR = 112
GP = 208
NPIX = GR * GP
ROW_VREGS = GP // 16
WRAD = 7
WROWS = 2 * WRAD + 1
OUT_ROWS = 304


def _prep_body(cls_ref, ctr_ref, reg_ref, cen_ref, thr_ref,
               s_ref, x1_ref, y1_ref, x2_ref, y2_ref, ids_ref,
               rmax_ref, ridx_ref):
    thr = thr_ref[0, 0]

    def cls_step(c, carry):
        acc, amax = carry
        x = cls_ref[c]
        gt = x > acc
        acc = jnp.where(gt, x, acc)
        amax = jnp.where(gt, c, amax)
        return acc, amax

    acc0 = cls_ref[0]
    amax0 = jnp.zeros((GR, GP), jnp.int32)
    cls_scores, cls_ids = jax.lax.fori_loop(1, NUM_CLASSES, cls_step, (acc0, amax0))

    score = cls_scores * ctr_ref[...]
    gi = jax.lax.broadcasted_iota(jnp.int32, (GR, GP), 0)
    ci = jax.lax.broadcasted_iota(jnp.int32, (GR, GP), 1)
    valid = (gi < H) & (ci < W)
    s = jnp.where((score > thr) & valid, score, NEG_INF)
    s_ref[...] = s

    rmax = jnp.max(s, axis=1, keepdims=True)
    minc = jnp.min(jnp.where(s == rmax, ci, jnp.int32(2**30)), axis=1,
                   keepdims=True)
    rowi = jax.lax.broadcasted_iota(jnp.int32, (GR, 1), 0)
    rmax_ref[...] = rmax
    ridx_ref[...] = (rowi * W + minc).astype(jnp.float32)

    x1_ref[...] = cen_ref[0] - reg_ref[0]
    y1_ref[...] = cen_ref[1] - reg_ref[1]
    x2_ref[...] = cen_ref[0] + reg_ref[2]
    y2_ref[...] = cen_ref[1] + reg_ref[3]
    ids_ref[...] = cls_ids.astype(jnp.float32)


def _nms_sc_body(s_hbm, x1_hbm, y1_hbm, x2_hbm, y2_hbm, rm_hbm, ri_hbm,
                 par_hbm, out_hbm,
                 s_v, x1_v, y1_v, x2_v, y2_v,
                 par_v, out_v, rmax_v, ridx_v, sem):
    cid = lax.axis_index("c")
    sid = lax.axis_index("s")

    @pl.when(jnp.logical_and(cid == 0, sid == 0))
    def _body():
        cps = [pltpu.async_copy(src, dst, sem) for src, dst in
               [(s_hbm, s_v), (x1_hbm, x1_v), (y1_hbm, y1_v),
                (x2_hbm, x2_v), (y2_hbm, y2_v)]]
        pltpu.sync_copy(rm_hbm, rmax_v)
        pltpu.sync_copy(ri_hbm, ridx_v)
        pltpu.sync_copy(par_hbm, par_v)
        for cp in cps:
            cp.wait()

        li = lax.iota(jnp.int32, 16)
        lif = li.astype(jnp.float32)
        iou_thr = par_v[...]
        zero16 = jnp.zeros((16,), jnp.int32)

        def scan_row(r):
            g_orig = r * W
            rb = r * GP
            acc = s_v[pl.ds(rb, 16)]
            idxv = jnp.float32(1.0) * g_orig + lif
            for t in range(1, ROW_VREGS):
                sv = s_v[pl.ds(rb + t * 16, 16)]
                gt = sv > acc
                acc = jnp.where(gt, sv, acc)
                idxv = jnp.where(gt, (jnp.float32(1.0) * g_orig + t * 16) + lif,
                                 idxv)
            rm = jnp.max(acc)
            ri = jnp.min(jnp.where(acc == rm, idxv, BIG_F))
            return rm, ri

        def step(k, carry):
            rmxs = [rmax_v[pl.ds(t * 16, 16)] for t in range(7)]
            rixs = [ridx_v[pl.ds(t * 16, 16)] for t in range(7)]
            t01 = jnp.maximum(rmxs[0], rmxs[1])
            t23 = jnp.maximum(rmxs[2], rmxs[3])
            t45 = jnp.maximum(rmxs[4], rmxs[5])
            m = jnp.max(jnp.maximum(jnp.maximum(t01, t23),
                                    jnp.maximum(t45, rmxs[6])))
            cand = jnp.full((16,), BIG_F, jnp.float32)
            for t in range(7):
                cand = jnp.minimum(cand, jnp.where(rmxs[t] == m, rixs[t], BIG_F))
            idxf = jnp.min(cand)

            widx = idxf.astype(jnp.int32)
            wg = widx // W
            wc = widx % W
            wpos = wg * GP + wc
            wp16 = zero16 + wpos
            wx1 = plsc.load_gather(x1_v, [wp16])
            wy1 = plsc.load_gather(y1_v, [wp16])
            wx2 = plsc.load_gather(x2_v, [wp16])
            wy2 = plsc.load_gather(y2_v, [wp16])
            war = (wx2 - wx1) * (wy2 - wy1)

            valid = m > NEG_INF
            vf = jnp.where(valid, 1.0, 0.0)
            outrec = jnp.where(li == 0, m * vf,
                     jnp.where(li == 1, idxf, jnp.where(li == 2, vf, 0.0)))
            out_v[pl.ds(k * 16, 16)] = outrec

            rows = wg - WRAD + li
            inbv = (rows >= 0) & (rows < GR) & (li < WROWS)
            rowsc = jnp.clip(rows, 0, GR - 1)
            crm = plsc.load_gather(rmax_v, [rowsc])
            cri = plsc.load_gather(ridx_v, [rowsc])
            cpos = jnp.clip(rowsc * GP + cri.astype(jnp.int32) - rowsc * W,
                            0, NPIX - 1)
            cx1 = plsc.load_gather(x1_v, [cpos])
            cy1 = plsc.load_gather(y1_v, [cpos])
            cx2 = plsc.load_gather(x2_v, [cpos])
            cy2 = plsc.load_gather(y2_v, [cpos])
            cinter = (jnp.maximum(jnp.minimum(wx2, cx2) - jnp.maximum(wx1, cx1), 0.0)
                      * jnp.maximum(jnp.minimum(wy2, cy2) - jnp.maximum(wy1, cy1), 0.0))
            ciou = cinter / (war + (cx2 - cx1) * (cy2 - cy1) - cinter + 1e-8)
            dead0 = (((ciou > iou_thr) | (cpos == wpos)) & (crm > NEG_INF)
                     & inbv).astype(jnp.int32)

            cl = jnp.maximum(wc - WRAD, 0)
            t0 = cl // 16
            ta = t0 * 16
            tb = jnp.minimum(t0 + 1, ROW_VREGS - 1) * 16
            slots = []
            for j in range(WROWS):
                row = wg - WRAD + j
                inb = jnp.logical_and(row >= 0, row < GR)
                rb = jnp.clip(row, 0, GR - 1) * GP
                for toff in (ta, tb):
                    beg = rb + toff
                    sl = pl.ds(beg, 16)
                    slots.append((sl, s_v[sl], x1_v[sl], y1_v[sl],
                                  x2_v[sl], y2_v[sl], beg, inb))
            for sl, sv, bx1, by1, bx2, by2, beg, inb in slots:
                ix1 = jnp.maximum(wx1, bx1)
                iy1 = jnp.maximum(wy1, by1)
                ix2 = jnp.minimum(wx2, bx2)
                iy2 = jnp.minimum(wy2, by2)
                inter = (jnp.maximum(ix2 - ix1, 0.0)
                         * jnp.maximum(iy2 - iy1, 0.0))
                areab = (bx2 - bx1) * (by2 - by1)
                iou = inter / (war + areab - inter + 1e-8)
                supp = ((iou > iou_thr) & inb) | ((beg + li) == wpos)
                s_v[sl] = jnp.where(supp, NEG_INF, sv)

            def any_dead(dead):
                return jnp.max(dead) == 1

            def rescan_one(dead):
                lane = jnp.min(jnp.where(dead == 1, li, 16))
                r = jnp.clip(wg - WRAD + lane, 0, GR - 1)
                rm, ri = scan_row(r)
                plsc.store_scatter(rmax_v, [zero16 + r], zero16 * 0.0 + rm,
                                   mask=li == 0)
                plsc.store_scatter(ridx_v, [zero16 + r], zero16 * 0.0 + ri,
                                   mask=li == 0)
                return jnp.where(li == lane, 0, dead)

            lax.while_loop(any_dead, rescan_one, dead0)
            return carry

        lax.fori_loop(0, MAX_OUT, step, 0, unroll=False)

        pltpu.sync_copy(out_v, out_hbm)


@jax.jit
def _decode_nms(cls_t, ctr_t, reg_t, centers, score_threshold, iou_threshold):
    def pitch(a):
        padw = [(0, 0)] * (a.ndim - 2) + [(0, GR - H), (0, GP - W)]
        return jnp.pad(a, padw)

    cls_p = pitch(cls_t[0].T.reshape(NUM_CLASSES, H, W))
    ctr_p = pitch(ctr_t[0].reshape(H, W))
    reg_p = pitch(reg_t[0].T.reshape(4, H, W))
    cen_p = pitch(centers.T.reshape(2, H, W))
    thr = jnp.asarray(score_threshold, jnp.float32).reshape(1, 1)

    grid2d = jax.ShapeDtypeStruct((GR, GP), jnp.float32)
    col1 = jax.ShapeDtypeStruct((GR, 1), jnp.float32)
    s0, x1, y1, x2, y2, idsf, rm0, ri0 = pl.pallas_call(
        _prep_body,
        out_shape=[grid2d] * 6 + [col1] * 2,
        in_specs=[
            pl.BlockSpec(memory_space=pltpu.VMEM),
            pl.BlockSpec(memory_space=pltpu.VMEM),
            pl.BlockSpec(memory_space=pltpu.VMEM),
            pl.BlockSpec(memory_space=pltpu.VMEM),
            pl.BlockSpec(memory_space=pltpu.SMEM),
        ],
        out_specs=[pl.BlockSpec(memory_space=pltpu.VMEM)] * 8,
    )(cls_p, ctr_p, reg_p, cen_p, thr)

    par = jnp.full((16,), jnp.asarray(iou_threshold, jnp.float32))

    nms = pl.kernel(
        _nms_sc_body,
        out_type=jax.ShapeDtypeStruct((OUT_ROWS * 16,), jnp.float32),
        mesh=plsc.VectorSubcoreMesh(core_axis_name="c", subcore_axis_name="s"),
        compiler_params=pltpu.CompilerParams(needs_layout_passes=False),
        scratch_types=[
            pltpu.VMEM((NPIX,), jnp.float32),
            pltpu.VMEM((NPIX,), jnp.float32),
            pltpu.VMEM((NPIX,), jnp.float32),
            pltpu.VMEM((NPIX,), jnp.float32),
            pltpu.VMEM((NPIX,), jnp.float32),
            pltpu.VMEM((16,), jnp.float32),
            pltpu.VMEM((OUT_ROWS * 16,), jnp.float32),
            pltpu.VMEM((GR,), jnp.float32),
            pltpu.VMEM((GR,), jnp.float32),
            pltpu.SemaphoreType.DMA,
        ],
    )

    out = nms(s0.reshape(NPIX), x1.reshape(NPIX), y1.reshape(NPIX),
              x2.reshape(NPIX), y2.reshape(NPIX),
              rm0.reshape(GR), ri0.reshape(GR), par)

    sel = out.reshape(OUT_ROWS, 16)[:MAX_OUT]
    out_scores = sel[:, 0][None]
    widx = sel[:, 1].astype(jnp.int32)
    vmask = sel[:, 2] > 0.0
    pidx = (widx // W) * GP + widx % W
    bx = jnp.stack([x1.reshape(NPIX)[pidx], y1.reshape(NPIX)[pidx],
                    x2.reshape(NPIX)[pidx], y2.reshape(NPIX)[pidx]], axis=-1)
    out_boxes = jnp.where(vmask[:, None], bx, 0.0)[None]
    out_ids = jnp.where(vmask, idsf.reshape(NPIX)[pidx].astype(jnp.int32), -1)[None]
    return out_boxes, out_scores, out_ids


def kernel(cls_target, ctr_target, reg_target, centers, score_threshold, iou_threshold):
    return _decode_nms(cls_target, ctr_target, reg_target, centers,
                       score_threshold, iou_threshold)
